# baseline (speedup 1.0000x reference)
import functools

import jax
import jax.numpy as jnp
from jax import lax
from jax.experimental import pallas as pl
from jax.experimental.pallas import tpu as pltpu
from jax.experimental.pallas import tpu_sc as plsc

N, E, D = 10000, 320000, 128
NC, NS = 2, 16
NW = NC * NS
EW = E // NW
C = 184
NCHUNK = EW // C
TAILC = EW - NCHUNK * C
NBUF = 2
ROWS_PER_SUB = 624
TAIL_BASE = ROWS_PER_SUB * NS
TAIL_ROWS = N - TAIL_BASE


def _sc_agg_body(h_hbm, src_hbm, dst_hbm, zeros_hbm, out_hbm,
                 sidx0, sidx1, sidx2, sidx3, didx0, didx1, didx2, didx3,
                 rows0, rows1, sidx_t, didx_t, acc,
                 gsem0, gsem1, ssem0, ssem1,
                 isem0, isem1, isem2, isem3):
    c = lax.axis_index("c")
    s = lax.axis_index("s")
    wid = s * NC + c
    base_w = wid * EW
    sidx = [sidx0, sidx1, sidx2, sidx3]
    didx = [didx0, didx1, didx2, didx3]
    rows = [rows0, rows1]
    gsem = [gsem0, gsem1]
    ssem = [ssem0, ssem1]
    isem = [isem0, isem1, isem2, isem3]

    def idx_start(i, m):
        base = base_w + i * C
        pltpu.async_copy(src_hbm.at[pl.ds(base, C)], sidx[m], isem[m])
        pltpu.async_copy(dst_hbm.at[pl.ds(base, C)], didx[m], isem[m])

    def idx_wait(i, m):
        base = base_w + i * C
        pltpu.make_async_copy(src_hbm.at[pl.ds(base, C)], sidx[m],
                              isem[m]).wait()
        pltpu.make_async_copy(dst_hbm.at[pl.ds(base, C)], didx[m],
                              isem[m]).wait()

    def gather_start(m, r):
        pltpu.async_copy(h_hbm.at[sidx[m]], rows[r], gsem[r])

    def scatter_prev(m, q):
        pltpu.make_async_copy(h_hbm.at[sidx[m]], rows[q], gsem[q]).wait()
        pltpu.async_copy(rows[q], acc.at[didx[m]], ssem[q], add=True)

    def drain_scatter(m, q):
        pltpu.make_async_copy(rows[q], acc.at[didx[m]], ssem[q]).wait()

    idx_start(0, 0)
    idx_start(1, 1)

    pltpu.sync_copy(zeros_hbm.at[pl.ds(s * ROWS_PER_SUB, ROWS_PER_SUB)],
                    acc.at[pl.ds(s * ROWS_PER_SUB, ROWS_PER_SUB)])

    @pl.when(s == 0)
    def _():
        pltpu.sync_copy(zeros_hbm.at[pl.ds(TAIL_BASE, TAIL_ROWS)],
                        acc.at[pl.ds(TAIL_BASE, TAIL_ROWS)])

    plsc.subcore_barrier()

    idx_wait(0, 0)
    gather_start(0, 0)
    idx_start(2, 2)
    idx_wait(1, 1)
    gather_start(1, 1)
    idx_start(3, 3)
    scatter_prev(0, 0)

    def substep(i, m, r, prefetch):
        q = 1 - r
        drain_scatter((m + 2) % 4, r)
        idx_wait(i, m)
        gather_start(m, r)
        if prefetch:
            idx_start(i + 2, (m + 2) % 4)
        scatter_prev((m + 3) % 4, q)

    def body(k, carry):
        i0 = 2 + 4 * k
        substep(i0, 2, 0, True)
        substep(i0 + 1, 3, 1, True)
        substep(i0 + 2, 0, 0, True)
        substep(i0 + 3, 1, 1, True)
        return carry

    lax.fori_loop(0, (NCHUNK - 6) // 4, body, 0)

    substep(NCHUNK - 4, 2, 0, True)
    substep(NCHUNK - 3, 3, 1, True)
    substep(NCHUNK - 2, 0, 0, False)
    substep(NCHUNK - 1, 1, 1, False)

    scatter_prev(1, 1)
    drain_scatter(0, 0)
    drain_scatter(1, 1)

    base = base_w + NCHUNK * C
    pltpu.sync_copy(src_hbm.at[pl.ds(base, TAILC)], sidx_t)
    pltpu.sync_copy(dst_hbm.at[pl.ds(base, TAILC)], didx_t)
    pltpu.async_copy(h_hbm.at[sidx_t], rows0.at[pl.ds(0, TAILC)],
                     gsem0).wait()
    pltpu.sync_copy(rows0.at[pl.ds(0, TAILC)], acc.at[didx_t], add=True)

    plsc.subcore_barrier()
    pltpu.sync_copy(acc.at[pl.ds(s * ROWS_PER_SUB, ROWS_PER_SUB)],
                    out_hbm.at[c, pl.ds(s * ROWS_PER_SUB, ROWS_PER_SUB)])

    @pl.when(s == 0)
    def _():
        pltpu.sync_copy(acc.at[pl.ds(TAIL_BASE, TAIL_ROWS)],
                        out_hbm.at[c, pl.ds(TAIL_BASE, TAIL_ROWS)])


_sc_agg = functools.partial(
    pl.kernel,
    out_type=jax.ShapeDtypeStruct((NC, N, D), jnp.float32),
    mesh=plsc.VectorSubcoreMesh(core_axis_name="c", subcore_axis_name="s"),
    scratch_types=(
        [pltpu.VMEM((C,), jnp.int32)] * 8
        + [pltpu.VMEM((C, D), jnp.float32)] * 2
        + [pltpu.VMEM((TAILC,), jnp.int32)] * 2
        + [pltpu.VMEM_SHARED((N, D), jnp.float32)]
        + [pltpu.SemaphoreType.DMA] * 8
    ),
)(_sc_agg_body)


def _tc_layer_bn_body(h_ref, p_ref, w_ref, b_ref, g_ref, be_ref, o_ref):
    z = h_ref[...] + p_ref[0] + p_ref[1]
    y = jnp.dot(z, w_ref[...], preferred_element_type=jnp.float32) + b_ref[...]
    m = jnp.mean(y, axis=0, keepdims=True)
    v = jnp.mean(y * y, axis=0, keepdims=True) - m * m
    yn = g_ref[...] * (y - m) * lax.rsqrt(v + 1e-5) + be_ref[...]
    o_ref[...] = jnp.maximum(yn, 0.0)


def _tc_layer_plain_body(h_ref, p_ref, w_ref, b_ref, o_ref):
    z = h_ref[...] + p_ref[0] + p_ref[1]
    o_ref[...] = (jnp.dot(z, w_ref[...], preferred_element_type=jnp.float32)
                  + b_ref[...])


def _tc_layer_bn(h, p, w, b, g, be):
    return pl.pallas_call(
        _tc_layer_bn_body,
        out_shape=jax.ShapeDtypeStruct((N, D), jnp.float32),
    )(h, p, w, b.reshape(1, D), g.reshape(1, D), be.reshape(1, D))


def _tc_layer_plain(h, p, w, b):
    return pl.pallas_call(
        _tc_layer_plain_body,
        out_shape=jax.ShapeDtypeStruct((N, D), jnp.float32),
    )(h, p, w, b.reshape(1, D))


def kernel(x, edge_index, W1, b1, g1, be1, W2, b2, g2, be2, W3, b3):
    src = edge_index[0]
    dst = edge_index[1]
    zeros = jnp.zeros_like(x)
    p = _sc_agg(x, src, dst, zeros)
    h = _tc_layer_bn(x, p, W1, b1, g1, be1)
    p = _sc_agg(h, src, dst, zeros)
    h = _tc_layer_bn(h, p, W2, b2, g2, be2)
    p = _sc_agg(h, src, dst, zeros)
    return _tc_layer_plain(h, p, W3, b3)


# speedup vs baseline: 1.0142x; 1.0142x over previous
import functools

import jax
import jax.numpy as jnp
from jax import lax
from jax.experimental import pallas as pl
from jax.experimental.pallas import tpu as pltpu
from jax.experimental.pallas import tpu_sc as plsc

N, E, D = 10000, 320000, 128
NC, NS = 2, 16
NW = NC * NS
EW = E // NW
C = 184
NCHUNK = EW // C
TAILC = EW - NCHUNK * C
NBUF = 2
ROWS_PER_SUB = 624
TAIL_BASE = ROWS_PER_SUB * NS
TAIL_ROWS = N - TAIL_BASE


def _sc_agg_body(h_hbm, src_hbm, dst_hbm, zeros_hbm, out_hbm,
                 sidx0, sidx1, sidx2, sidx3, didx0, didx1, didx2, didx3,
                 rows0, rows1, sidx_t, didx_t, acc,
                 gsem0, gsem1, ssem0, ssem1,
                 isem0, isem1, isem2, isem3):
    c = lax.axis_index("c")
    s = lax.axis_index("s")
    wid = s * NC + c
    base_w = wid * EW
    sidx = [sidx0, sidx1, sidx2, sidx3]
    didx = [didx0, didx1, didx2, didx3]
    rows = [rows0, rows1]
    gsem = [gsem0, gsem1]
    ssem = [ssem0, ssem1]
    isem = [isem0, isem1, isem2, isem3]

    def idx_start(i, m):
        base = base_w + i * C
        pltpu.async_copy(src_hbm.at[pl.ds(base, C)], sidx[m], isem[m])
        pltpu.async_copy(dst_hbm.at[pl.ds(base, C)], didx[m], isem[m])

    def idx_wait(i, m):
        base = base_w + i * C
        pltpu.make_async_copy(src_hbm.at[pl.ds(base, C)], sidx[m],
                              isem[m]).wait()
        pltpu.make_async_copy(dst_hbm.at[pl.ds(base, C)], didx[m],
                              isem[m]).wait()

    def gather_start(m, r):
        pltpu.async_copy(h_hbm.at[sidx[m]], rows[r], gsem[r])

    def scatter_prev(m, q):
        pltpu.make_async_copy(h_hbm.at[sidx[m]], rows[q], gsem[q]).wait()
        pltpu.async_copy(rows[q], acc.at[didx[m]], ssem[q], add=True)

    def drain_scatter(m, q):
        pltpu.make_async_copy(rows[q], acc.at[didx[m]], ssem[q]).wait()

    idx_start(0, 0)
    idx_start(1, 1)

    @pl.when(c == 0)
    def _():
        pltpu.sync_copy(h_hbm.at[pl.ds(s * ROWS_PER_SUB, ROWS_PER_SUB)],
                        acc.at[pl.ds(s * ROWS_PER_SUB, ROWS_PER_SUB)])

    @pl.when(c == 1)
    def _():
        pltpu.sync_copy(zeros_hbm.at[pl.ds(s * ROWS_PER_SUB, ROWS_PER_SUB)],
                        acc.at[pl.ds(s * ROWS_PER_SUB, ROWS_PER_SUB)])

    @pl.when((s == 0) & (c == 0))
    def _():
        pltpu.sync_copy(h_hbm.at[pl.ds(TAIL_BASE, TAIL_ROWS)],
                        acc.at[pl.ds(TAIL_BASE, TAIL_ROWS)])

    @pl.when((s == 0) & (c == 1))
    def _():
        pltpu.sync_copy(zeros_hbm.at[pl.ds(TAIL_BASE, TAIL_ROWS)],
                        acc.at[pl.ds(TAIL_BASE, TAIL_ROWS)])

    plsc.subcore_barrier()

    idx_wait(0, 0)
    gather_start(0, 0)
    idx_start(2, 2)
    idx_wait(1, 1)
    gather_start(1, 1)
    idx_start(3, 3)
    scatter_prev(0, 0)

    def substep(i, m, r, prefetch):
        q = 1 - r
        drain_scatter((m + 2) % 4, r)
        idx_wait(i, m)
        gather_start(m, r)
        if prefetch:
            idx_start(i + 2, (m + 2) % 4)
        scatter_prev((m + 3) % 4, q)

    def body(k, carry):
        i0 = 2 + 4 * k
        substep(i0, 2, 0, True)
        substep(i0 + 1, 3, 1, True)
        substep(i0 + 2, 0, 0, True)
        substep(i0 + 3, 1, 1, True)
        return carry

    lax.fori_loop(0, (NCHUNK - 6) // 4, body, 0)

    substep(NCHUNK - 4, 2, 0, True)
    substep(NCHUNK - 3, 3, 1, True)
    substep(NCHUNK - 2, 0, 0, False)
    substep(NCHUNK - 1, 1, 1, False)

    scatter_prev(1, 1)
    drain_scatter(0, 0)
    drain_scatter(1, 1)

    base = base_w + NCHUNK * C
    pltpu.sync_copy(src_hbm.at[pl.ds(base, TAILC)], sidx_t)
    pltpu.sync_copy(dst_hbm.at[pl.ds(base, TAILC)], didx_t)
    pltpu.async_copy(h_hbm.at[sidx_t], rows0.at[pl.ds(0, TAILC)],
                     gsem0).wait()
    pltpu.sync_copy(rows0.at[pl.ds(0, TAILC)], acc.at[didx_t], add=True)

    plsc.subcore_barrier()
    pltpu.sync_copy(acc.at[pl.ds(s * ROWS_PER_SUB, ROWS_PER_SUB)],
                    out_hbm.at[c, pl.ds(s * ROWS_PER_SUB, ROWS_PER_SUB)])

    @pl.when(s == 0)
    def _():
        pltpu.sync_copy(acc.at[pl.ds(TAIL_BASE, TAIL_ROWS)],
                        out_hbm.at[c, pl.ds(TAIL_BASE, TAIL_ROWS)])


_sc_agg = functools.partial(
    pl.kernel,
    out_type=jax.ShapeDtypeStruct((NC, N, D), jnp.float32),
    mesh=plsc.VectorSubcoreMesh(core_axis_name="c", subcore_axis_name="s"),
    scratch_types=(
        [pltpu.VMEM((C,), jnp.int32)] * 8
        + [pltpu.VMEM((C, D), jnp.float32)] * 2
        + [pltpu.VMEM((TAILC,), jnp.int32)] * 2
        + [pltpu.VMEM_SHARED((N, D), jnp.float32)]
        + [pltpu.SemaphoreType.DMA] * 8
    ),
)(_sc_agg_body)


def _tc_layer_bn_body(p_ref, w_ref, b_ref, g_ref, be_ref, o_ref):
    z = p_ref[0] + p_ref[1]
    y = jnp.dot(z, w_ref[...], preferred_element_type=jnp.float32) + b_ref[...]
    m = jnp.mean(y, axis=0, keepdims=True)
    v = jnp.mean(y * y, axis=0, keepdims=True) - m * m
    yn = g_ref[...] * (y - m) * lax.rsqrt(v + 1e-5) + be_ref[...]
    o_ref[...] = jnp.maximum(yn, 0.0)


def _tc_layer_plain_body(p_ref, w_ref, b_ref, o_ref):
    z = p_ref[0] + p_ref[1]
    o_ref[...] = (jnp.dot(z, w_ref[...], preferred_element_type=jnp.float32)
                  + b_ref[...])


def _tc_layer_bn(p, w, b, g, be):
    return pl.pallas_call(
        _tc_layer_bn_body,
        out_shape=jax.ShapeDtypeStruct((N, D), jnp.float32),
    )(p, w, b.reshape(1, D), g.reshape(1, D), be.reshape(1, D))


def _tc_layer_plain(p, w, b):
    return pl.pallas_call(
        _tc_layer_plain_body,
        out_shape=jax.ShapeDtypeStruct((N, D), jnp.float32),
    )(p, w, b.reshape(1, D))


def kernel(x, edge_index, W1, b1, g1, be1, W2, b2, g2, be2, W3, b3):
    src = edge_index[0]
    dst = edge_index[1]
    zeros = jnp.zeros_like(x)
    p = _sc_agg(x, src, dst, zeros)
    h = _tc_layer_bn(p, W1, b1, g1, be1)
    p = _sc_agg(h, src, dst, zeros)
    h = _tc_layer_bn(p, W2, b2, g2, be2)
    p = _sc_agg(h, src, dst, zeros)
    return _tc_layer_plain(p, W3, b3)
